# Initial kernel scaffold; baseline (speedup 1.0000x reference)
import jax
import jax.numpy as jnp
from jax.experimental import pallas as pl


def kernel(X, adj_t, tuples_coo, W, ln_scale, ln_bias):
    raise NotImplementedError("write your pallas kernel here")


# speedup vs baseline: 1.7936x; 1.7936x over previous
import functools

import jax
import jax.numpy as jnp
from jax import lax
from jax.experimental import pallas as pl
from jax.experimental.pallas import tpu as pltpu
from jax.experimental.pallas import tpu_sc as plsc

N = 4096
D = 256
T = 32768
OUT = 256
KSEL = 8
BMAX = 16

BLK = 256
NI = N // BLK
NK = N // BLK


def _deg_xw_body(adj_ref, x_ref, w0_ref, deg_ref, xw_ref):
    deg_ref[0, 0, :] = jnp.sum(adj_ref[...], axis=1)
    xw_ref[...] = jnp.dot(x_ref[...], w0_ref[...],
                          preferred_element_type=jnp.float32)


def _deg_xw(adj_t, X, W0):
    return pl.pallas_call(
        _deg_xw_body,
        grid=(NI,),
        in_specs=[
            pl.BlockSpec((BLK, N), lambda i: (i, 0)),
            pl.BlockSpec((BLK, D), lambda i: (i, 0)),
            pl.BlockSpec((D, OUT), lambda i: (0, 0)),
        ],
        out_specs=[
            pl.BlockSpec((1, 1, BLK), lambda i: (i, 0, 0)),
            pl.BlockSpec((BLK, OUT), lambda i: (i, 0)),
        ],
        out_shape=[
            jax.ShapeDtypeStruct((NI, 1, BLK), jnp.float32),
            jax.ShapeDtypeStruct((N, OUT), jnp.float32),
        ],
    )(adj_t, X, W0)


def _topk_body(deg_ref, idx_ref, scale_ref):
    d = deg_ref[...]
    gid = (lax.broadcasted_iota(jnp.int32, d.shape, 0) * 128
           + lax.broadcasted_iota(jnp.int32, d.shape, 1))
    cur = d
    vals = []
    for j in range(BMAX):
        m = jnp.max(cur)
        ix = jnp.min(jnp.where(cur == m, gid, jnp.int32(2**30)))
        vals.append(m)
        idx_ref[j] = ix
        cur = jnp.where(gid == ix, -jnp.inf, cur)
    b_count = jnp.int32(KSEL)
    for j in range(KSEL, BMAX):
        b_count = b_count + (vals[j] == vals[KSEL - 1]).astype(jnp.int32)
    inv_sqrt_b = lax.rsqrt(b_count.astype(jnp.float32))
    for b in range(BMAX):
        scale_ref[b] = jnp.where(b < b_count, inv_sqrt_b, 0.0)


def _topk(deg):
    return pl.pallas_call(
        _topk_body,
        in_specs=[pl.BlockSpec(memory_space=pltpu.VMEM)],
        out_specs=[
            pl.BlockSpec(memory_space=pltpu.SMEM),
            pl.BlockSpec(memory_space=pltpu.SMEM),
        ],
        out_shape=[
            jax.ShapeDtypeStruct((BMAX,), jnp.int32),
            jax.ShapeDtypeStruct((BMAX,), jnp.float32),
        ],
    )(deg)


def _table_body(idx_ref, scale_ref, adj_ref, xw_ref, wrow_ref, lns_ref,
                lnb_ref, table_ref, acc_ref, accg_ref):
    k = pl.program_id(1)

    @pl.when(k == 0)
    def _():
        acc_ref[...] = jnp.zeros_like(acc_ref)
        accg_ref[...] = jnp.zeros_like(accg_ref)

    adj = adj_ref[...]
    acc_ref[...] += jnp.dot(adj, xw_ref[...],
                            preferred_element_type=jnp.float32)

    rowid = k * BLK + lax.broadcasted_iota(jnp.int32, (BLK, BMAX), 0)
    colid = lax.broadcasted_iota(jnp.int32, (BLK, BMAX), 1)
    e = jnp.zeros((BLK, BMAX), jnp.float32)
    for b in range(BMAX):
        e += jnp.where((colid == b) & (rowid == idx_ref[b]), 1.0, 0.0)
    accg_ref[...] += jnp.dot(adj, e, preferred_element_type=jnp.float32)

    @pl.when(k == NK - 1)
    def _():
        a = acc_ref[...]
        g = accg_ref[...]
        w = wrow_ref[...]
        lns = lns_ref[...]
        lnb = lnb_ref[...]
        for b in range(BMAX):
            h = jnp.maximum(a + g[:, b:b + 1] * w, 0.0)
            mu = jnp.mean(h, axis=1, keepdims=True)
            var = jnp.mean((h - mu) ** 2, axis=1, keepdims=True)
            holo = (h - mu) * lax.rsqrt(var + 1e-5) * lns + lnb
            table_ref[:, b * OUT:(b + 1) * OUT] = holo * scale_ref[b]


def _table(idx, scales, adj_t, xw, wrow, lns, lnb):
    return pl.pallas_call(
        _table_body,
        grid=(NI, NK),
        in_specs=[
            pl.BlockSpec(memory_space=pltpu.SMEM),
            pl.BlockSpec(memory_space=pltpu.SMEM),
            pl.BlockSpec((BLK, BLK), lambda i, k: (i, k)),
            pl.BlockSpec((BLK, OUT), lambda i, k: (k, 0)),
            pl.BlockSpec((1, OUT), lambda i, k: (0, 0)),
            pl.BlockSpec((1, OUT), lambda i, k: (0, 0)),
            pl.BlockSpec((1, OUT), lambda i, k: (0, 0)),
        ],
        out_specs=pl.BlockSpec((BLK, BMAX * OUT), lambda i, k: (i, 0)),
        out_shape=jax.ShapeDtypeStruct((N, BMAX * OUT), jnp.float32),
        scratch_shapes=[
            pltpu.VMEM((BLK, OUT), jnp.float32),
            pltpu.VMEM((BLK, BMAX), jnp.float32),
        ],
    )(idx, scales, adj_t, xw, wrow, lns, lnb)


NW = 32
TPW = T // NW
CH = 8
ROW = BMAX * OUT


def _sc_body(table_hbm, tup_hbm, out_hbm, idx_s, idx_d, srows, drows, orows,
             sem_s, sem_d):
    wid = lax.axis_index("s") * 2 + lax.axis_index("c")
    base = wid * TPW
    pltpu.sync_copy(tup_hbm.at[0, pl.ds(base, TPW)], idx_s)
    pltpu.sync_copy(tup_hbm.at[1, pl.ds(base, TPW)], idx_d)

    def chunk(c, carry):
        co = c * CH
        cp_s = pltpu.async_copy(table_hbm.at[idx_s.at[pl.ds(co, CH)]],
                                srows, sem_s)
        cp_d = pltpu.async_copy(table_hbm.at[idx_d.at[pl.ds(co, CH)]],
                                drows, sem_d)
        cp_s.wait()
        cp_d.wait()
        for t in range(CH):
            def col(oc, carry2):
                off = oc * 16
                acc = jnp.zeros((16,), jnp.float32)
                for b in range(BMAX):
                    sv = srows[t, pl.ds(b * OUT + off, 16)]
                    dv = drows[t, pl.ds(b * OUT + off, 16)]
                    acc = acc + sv * dv
                orows[t, pl.ds(off, 16)] = acc
                return carry2
            lax.fori_loop(0, OUT // 16, col, 0)
        pltpu.sync_copy(orows, out_hbm.at[pl.ds(base + co, CH)])
        return carry

    lax.fori_loop(0, TPW // CH, chunk, 0)


@functools.cache
def _sc_gather():
    return pl.kernel(
        _sc_body,
        out_type=jax.ShapeDtypeStruct((T, OUT), jnp.float32),
        mesh=plsc.VectorSubcoreMesh(core_axis_name="c", subcore_axis_name="s"),
        scratch_types=[
            pltpu.VMEM((TPW,), jnp.int32),
            pltpu.VMEM((TPW,), jnp.int32),
            pltpu.VMEM((CH, ROW), jnp.float32),
            pltpu.VMEM((CH, ROW), jnp.float32),
            pltpu.VMEM((CH, OUT), jnp.float32),
            pltpu.SemaphoreType.DMA,
            pltpu.SemaphoreType.DMA,
        ],
    )


def kernel(X, adj_t, tuples_coo, W, ln_scale, ln_bias):
    W0 = W[:D]
    wrow = W[D:D + 1]
    deg3, xw = _deg_xw(adj_t, X, W0)
    idx, scales = _topk(deg3.reshape(32, 128))
    table = _table(idx, scales, adj_t, xw, wrow,
                   ln_scale.reshape(1, OUT), ln_bias.reshape(1, OUT))
    return _sc_gather()(table, tuples_coo.astype(jnp.int32))


# speedup vs baseline: 2.2767x; 1.2693x over previous
import functools

import jax
import jax.numpy as jnp
from jax import lax
from jax.experimental import pallas as pl
from jax.experimental.pallas import tpu as pltpu
from jax.experimental.pallas import tpu_sc as plsc

N = 4096
D = 256
T = 32768
OUT = 256
KSEL = 8
BMAX = 16

BLK = 256
NI = N // BLK
NK = N // BLK


def _deg_xw_body(adj_ref, x_ref, w0_ref, deg_ref, xw_ref):
    deg_ref[0, 0, :] = jnp.sum(adj_ref[...], axis=1)
    xw_ref[...] = jnp.dot(x_ref[...], w0_ref[...],
                          preferred_element_type=jnp.float32)


def _deg_xw(adj_t, X, W0):
    return pl.pallas_call(
        _deg_xw_body,
        grid=(NI,),
        in_specs=[
            pl.BlockSpec((BLK, N), lambda i: (i, 0)),
            pl.BlockSpec((BLK, D), lambda i: (i, 0)),
            pl.BlockSpec((D, OUT), lambda i: (0, 0)),
        ],
        out_specs=[
            pl.BlockSpec((1, 1, BLK), lambda i: (i, 0, 0)),
            pl.BlockSpec((BLK, OUT), lambda i: (i, 0)),
        ],
        out_shape=[
            jax.ShapeDtypeStruct((NI, 1, BLK), jnp.float32),
            jax.ShapeDtypeStruct((N, OUT), jnp.float32),
        ],
    )(adj_t, X, W0)


def _topk_body(deg_ref, idx_ref, scale_ref):
    d = deg_ref[...]
    gid = (lax.broadcasted_iota(jnp.int32, d.shape, 0) * 128
           + lax.broadcasted_iota(jnp.int32, d.shape, 1))
    cur = d
    vals = []
    for j in range(BMAX):
        m = jnp.max(cur)
        ix = jnp.min(jnp.where(cur == m, gid, jnp.int32(2**30)))
        vals.append(m)
        idx_ref[j] = ix
        cur = jnp.where(gid == ix, -jnp.inf, cur)
    b_count = jnp.int32(KSEL)
    for j in range(KSEL, BMAX):
        b_count = b_count + (vals[j] == vals[KSEL - 1]).astype(jnp.int32)
    inv_sqrt_b = lax.rsqrt(b_count.astype(jnp.float32))
    for b in range(BMAX):
        scale_ref[b] = jnp.where(b < b_count, inv_sqrt_b, 0.0)


def _topk(deg):
    return pl.pallas_call(
        _topk_body,
        in_specs=[pl.BlockSpec(memory_space=pltpu.VMEM)],
        out_specs=[
            pl.BlockSpec(memory_space=pltpu.SMEM),
            pl.BlockSpec(memory_space=pltpu.SMEM),
        ],
        out_shape=[
            jax.ShapeDtypeStruct((BMAX,), jnp.int32),
            jax.ShapeDtypeStruct((BMAX,), jnp.float32),
        ],
    )(deg)


def _table_body(idx_ref, scale_ref, adj_ref, xw_ref, wrow_ref, lns_ref,
                lnb_ref, table_ref, acc_ref, accg_ref):
    k = pl.program_id(1)

    @pl.when(k == 0)
    def _():
        acc_ref[...] = jnp.zeros_like(acc_ref)
        accg_ref[...] = jnp.zeros_like(accg_ref)

    adj = adj_ref[...]
    acc_ref[...] += jnp.dot(adj, xw_ref[...],
                            preferred_element_type=jnp.float32)

    rowid = k * BLK + lax.broadcasted_iota(jnp.int32, (BLK, BMAX), 0)
    colid = lax.broadcasted_iota(jnp.int32, (BLK, BMAX), 1)
    e = jnp.zeros((BLK, BMAX), jnp.float32)
    for b in range(BMAX):
        e += jnp.where((colid == b) & (rowid == idx_ref[b]), 1.0, 0.0)
    accg_ref[...] += jnp.dot(adj, e, preferred_element_type=jnp.float32)

    @pl.when(k == NK - 1)
    def _():
        a = acc_ref[...]
        g = accg_ref[...]
        w = wrow_ref[...]
        lns = lns_ref[...]
        lnb = lnb_ref[...]
        for b in range(BMAX):
            h = jnp.maximum(a + g[:, b:b + 1] * w, 0.0)
            mu = jnp.mean(h, axis=1, keepdims=True)
            var = jnp.mean((h - mu) ** 2, axis=1, keepdims=True)
            holo = (h - mu) * lax.rsqrt(var + 1e-5) * lns + lnb
            table_ref[:, b * OUT:(b + 1) * OUT] = (
                holo * scale_ref[b]).astype(jnp.bfloat16)


def _table(idx, scales, adj_t, xw, wrow, lns, lnb):
    return pl.pallas_call(
        _table_body,
        grid=(NI, NK),
        in_specs=[
            pl.BlockSpec(memory_space=pltpu.SMEM),
            pl.BlockSpec(memory_space=pltpu.SMEM),
            pl.BlockSpec((BLK, BLK), lambda i, k: (i, k)),
            pl.BlockSpec((BLK, OUT), lambda i, k: (k, 0)),
            pl.BlockSpec((1, OUT), lambda i, k: (0, 0)),
            pl.BlockSpec((1, OUT), lambda i, k: (0, 0)),
            pl.BlockSpec((1, OUT), lambda i, k: (0, 0)),
        ],
        out_specs=pl.BlockSpec((BLK, BMAX * OUT), lambda i, k: (i, 0)),
        out_shape=jax.ShapeDtypeStruct((N, BMAX * OUT), jnp.bfloat16),
        scratch_shapes=[
            pltpu.VMEM((BLK, OUT), jnp.float32),
            pltpu.VMEM((BLK, BMAX), jnp.float32),
        ],
    )(idx, scales, adj_t, xw, wrow, lns, lnb)


NW = 32
TPW = T // NW
CH = 8
NCH = TPW // CH
ROW = BMAX * OUT
ROWW = ROW // 2
SL = ROWW // 128


def _sc_body(table_hbm, tup_hbm, out_hbm, idx_s, idx_d, sbuf0, sbuf1, dbuf0,
             dbuf1, orows, sem_s0, sem_s1, sem_d0, sem_d1):
    wid = lax.axis_index("s") * 2 + lax.axis_index("c")
    base = wid * TPW
    pltpu.sync_copy(tup_hbm.at[0, pl.ds(base, TPW)], idx_s)
    pltpu.sync_copy(tup_hbm.at[1, pl.ds(base, TPW)], idx_d)

    sbufs = (sbuf0, sbuf1)
    dbufs = (dbuf0, dbuf1)
    sems_s = (sem_s0, sem_s1)
    sems_d = (sem_d0, sem_d1)

    def fire(c, p):
        co = jnp.minimum(c, NCH - 1) * CH
        pltpu.async_copy(table_hbm.at[idx_s.at[pl.ds(co, CH)]],
                         sbufs[p], sems_s[p])
        pltpu.async_copy(table_hbm.at[idx_d.at[pl.ds(co, CH)]],
                         dbufs[p], sems_d[p])

    def wait(c, p):
        co = jnp.minimum(c, NCH - 1) * CH
        pltpu.make_async_copy(table_hbm.at[idx_s.at[pl.ds(co, CH)]],
                              sbufs[p], sems_s[p]).wait()
        pltpu.make_async_copy(table_hbm.at[idx_d.at[pl.ds(co, CH)]],
                              dbufs[p], sems_d[p]).wait()

    iot = lax.iota(jnp.int32, 16)

    def compute(p, co):
        buf_s = sbufs[p]
        buf_d = dbufs[p]

        def tup(t, carry):
            trow = jnp.zeros((16,), jnp.int32) + t
            for w in range(8):
                lo = w * 16
                acc_e = jnp.zeros((16,), jnp.float32)
                acc_o = jnp.zeros((16,), jnp.float32)
                for b in range(BMAX):
                    sv = plsc.bitcast(buf_s[t, b, pl.ds(lo, 16)],
                                      jnp.bfloat16)
                    dv = plsc.bitcast(buf_d[t, b, pl.ds(lo, 16)],
                                      jnp.bfloat16)
                    pe, po = plsc.unpack(
                        sv * dv, format=plsc.PackFormat.INTERLEAVED)
                    acc_e = acc_e + pe
                    acc_o = acc_o + po
                ce = iot * 2 + (w * 32)
                plsc.store_scatter(orows, [trow, ce], acc_e)
                plsc.store_scatter(orows, [trow, ce + 1], acc_o)
            return carry

        lax.fori_loop(0, CH, tup, 0)
        pltpu.sync_copy(orows, out_hbm.at[pl.ds(base + co, CH)])

    fire(0, 0)

    def pair(c2, carry):
        c0 = c2 * 2
        fire(c0 + 1, 1)
        wait(c0, 0)
        compute(0, c0 * CH)
        fire(c0 + 2, 0)
        wait(c0 + 1, 1)
        compute(1, (c0 + 1) * CH)
        return carry

    lax.fori_loop(0, NCH // 2, pair, 0)
    wait(NCH, 0)


@functools.cache
def _sc_gather():
    return pl.kernel(
        _sc_body,
        out_type=jax.ShapeDtypeStruct((T, OUT), jnp.float32),
        mesh=plsc.VectorSubcoreMesh(core_axis_name="c", subcore_axis_name="s"),
        compiler_params=pltpu.CompilerParams(needs_layout_passes=False),
        scratch_types=[
            pltpu.VMEM((TPW,), jnp.int32),
            pltpu.VMEM((TPW,), jnp.int32),
            pltpu.VMEM((CH, SL, 128), jnp.int32),
            pltpu.VMEM((CH, SL, 128), jnp.int32),
            pltpu.VMEM((CH, SL, 128), jnp.int32),
            pltpu.VMEM((CH, SL, 128), jnp.int32),
            pltpu.VMEM((CH, OUT), jnp.float32),
            pltpu.SemaphoreType.DMA,
            pltpu.SemaphoreType.DMA,
            pltpu.SemaphoreType.DMA,
            pltpu.SemaphoreType.DMA,
        ],
    )


def kernel(X, adj_t, tuples_coo, W, ln_scale, ln_bias):
    W0 = W[:D]
    wrow = W[D:D + 1]
    deg3, xw = _deg_xw(adj_t, X, W0)
    idx, scales = _topk(deg3.reshape(32, 128))
    table = _table(idx, scales, adj_t, xw, wrow,
                   ln_scale.reshape(1, OUT), ln_bias.reshape(1, OUT))
    table_w = lax.bitcast_convert_type(
        table.reshape(N, ROWW, 2), jnp.int32)
    return _sc_gather()(table_w.reshape(N, SL, 128),
                        tuples_coo.astype(jnp.int32))


# speedup vs baseline: 3.6178x; 1.5891x over previous
import functools

import jax
import jax.numpy as jnp
from jax import lax
from jax.experimental import pallas as pl
from jax.experimental.pallas import tpu as pltpu
from jax.experimental.pallas import tpu_sc as plsc

N = 4096
D = 256
T = 32768
OUT = 256
KSEL = 8
BMAX = 16

BLK = 256
NI = N // BLK
NK = N // BLK


def _deg_xw_body(adj_ref, x_ref, w0_ref, deg_ref, xw_ref, adjb_ref):
    adj = adj_ref[...]
    deg_ref[0, 0, :] = jnp.sum(adj, axis=1)
    adjb_ref[...] = adj.astype(jnp.bfloat16)
    xw_ref[...] = jnp.dot(x_ref[...], w0_ref[...],
                          preferred_element_type=jnp.float32
                          ).astype(jnp.bfloat16)


def _deg_xw(adj_t, X, W0):
    return pl.pallas_call(
        _deg_xw_body,
        grid=(NI,),
        in_specs=[
            pl.BlockSpec((BLK, N), lambda i: (i, 0)),
            pl.BlockSpec((BLK, D), lambda i: (i, 0)),
            pl.BlockSpec((D, OUT), lambda i: (0, 0)),
        ],
        out_specs=[
            pl.BlockSpec((1, 1, BLK), lambda i: (i, 0, 0)),
            pl.BlockSpec((BLK, OUT), lambda i: (i, 0)),
            pl.BlockSpec((BLK, N), lambda i: (i, 0)),
        ],
        out_shape=[
            jax.ShapeDtypeStruct((NI, 1, BLK), jnp.float32),
            jax.ShapeDtypeStruct((N, OUT), jnp.bfloat16),
            jax.ShapeDtypeStruct((N, N), jnp.bfloat16),
        ],
    )(adj_t, X, W0)


def _topk_body(deg_ref, idx_ref, scale_ref, e_ref):
    d = deg_ref[...]
    gid = (lax.broadcasted_iota(jnp.int32, d.shape, 0) * 128
           + lax.broadcasted_iota(jnp.int32, d.shape, 1))
    cur = d
    vals = []
    for j in range(BMAX):
        m = jnp.max(cur)
        ix = jnp.min(jnp.where(cur == m, gid, jnp.int32(2**30)))
        vals.append(m)
        idx_ref[j] = ix
        cur = jnp.where(gid == ix, -jnp.inf, cur)
    b_count = jnp.int32(KSEL)
    for j in range(KSEL, BMAX):
        b_count = b_count + (vals[j] == vals[KSEL - 1]).astype(jnp.int32)
    inv_sqrt_b = lax.rsqrt(b_count.astype(jnp.float32))
    for b in range(BMAX):
        scale_ref[b] = jnp.where(b < b_count, inv_sqrt_b, 0.0)
    colid = lax.broadcasted_iota(jnp.int32, (1, BMAX), 1)
    idxvec = jnp.zeros((1, BMAX), jnp.int32)
    for b in range(BMAX):
        idxvec = jnp.where(colid == b, idx_ref[b], idxvec)
    rowid = lax.broadcasted_iota(jnp.int32, (N, BMAX), 0)
    e_ref[...] = (rowid == idxvec).astype(jnp.bfloat16)


def _topk(deg):
    return pl.pallas_call(
        _topk_body,
        in_specs=[pl.BlockSpec(memory_space=pltpu.VMEM)],
        out_specs=[
            pl.BlockSpec(memory_space=pltpu.SMEM),
            pl.BlockSpec(memory_space=pltpu.SMEM),
            pl.BlockSpec(memory_space=pltpu.VMEM),
        ],
        out_shape=[
            jax.ShapeDtypeStruct((BMAX,), jnp.int32),
            jax.ShapeDtypeStruct((BMAX,), jnp.float32),
            jax.ShapeDtypeStruct((N, BMAX), jnp.bfloat16),
        ],
    )(deg)


def _pack_words(x):
    lo = lax.bitcast_convert_type(x[:, :128].astype(jnp.bfloat16),
                                  jnp.uint16).astype(jnp.uint32)
    hi = lax.bitcast_convert_type(x[:, 128:].astype(jnp.bfloat16),
                                  jnp.uint16).astype(jnp.uint32)
    return lax.bitcast_convert_type(lo | (hi << 16), jnp.int32)


def _table_body(scale_ref, adj_ref, xw_ref, e_ref, wrow_ref, lns_ref,
                lnb_ref, table_ref, acc_ref, accg_ref):
    k = pl.program_id(1)

    @pl.when(k == 0)
    def _():
        acc_ref[...] = jnp.zeros_like(acc_ref)
        accg_ref[...] = jnp.zeros_like(accg_ref)

    adj = adj_ref[...]
    acc_ref[...] += jnp.dot(adj, xw_ref[...],
                            preferred_element_type=jnp.float32)
    accg_ref[...] += jnp.dot(adj, e_ref[...],
                             preferred_element_type=jnp.float32)

    @pl.when(k == NK - 1)
    def _():
        a = acc_ref[...]
        g = accg_ref[...]
        w = wrow_ref[...]
        lns = lns_ref[...]
        lnb = lnb_ref[...]
        for b in range(BMAX):
            h = jnp.maximum(a + g[:, b:b + 1] * w, 0.0)
            mu = jnp.mean(h, axis=1, keepdims=True)
            var = jnp.mean((h - mu) ** 2, axis=1, keepdims=True)
            holo = (h - mu) * lax.rsqrt(var + 1e-5) * lns + lnb
            table_ref[:, b, :] = _pack_words(holo * scale_ref[b])


def _table(scales, adj_bf, xw, e, wrow, lns, lnb):
    return pl.pallas_call(
        _table_body,
        grid=(NI, NK),
        in_specs=[
            pl.BlockSpec(memory_space=pltpu.SMEM),
            pl.BlockSpec((BLK, BLK), lambda i, k: (i, k)),
            pl.BlockSpec((BLK, OUT), lambda i, k: (k, 0)),
            pl.BlockSpec((BLK, BMAX), lambda i, k: (k, 0)),
            pl.BlockSpec((1, OUT), lambda i, k: (0, 0)),
            pl.BlockSpec((1, OUT), lambda i, k: (0, 0)),
            pl.BlockSpec((1, OUT), lambda i, k: (0, 0)),
        ],
        out_specs=pl.BlockSpec((BLK, BMAX, 128), lambda i, k: (i, 0, 0)),
        out_shape=jax.ShapeDtypeStruct((N, BMAX, 128), jnp.int32),
        scratch_shapes=[
            pltpu.VMEM((BLK, OUT), jnp.float32),
            pltpu.VMEM((BLK, BMAX), jnp.float32),
        ],
    )(scales, adj_bf, xw, e, wrow, lns, lnb)


NW = 32
TPW = T // NW
CH = 8
NCH = TPW // CH


def _sc_body(table_hbm, tup_hbm, out_hbm, idx_s, idx_d, sbuf0, sbuf1, dbuf0,
             dbuf1, orows, sem_s0, sem_s1, sem_d0, sem_d1):
    wid = lax.axis_index("s") * 2 + lax.axis_index("c")
    base = wid * TPW
    pltpu.sync_copy(tup_hbm.at[0, pl.ds(base, TPW)], idx_s)
    pltpu.sync_copy(tup_hbm.at[1, pl.ds(base, TPW)], idx_d)

    sbufs = (sbuf0, sbuf1)
    dbufs = (dbuf0, dbuf1)
    sems_s = (sem_s0, sem_s1)
    sems_d = (sem_d0, sem_d1)

    def fire(c, p):
        co = jnp.minimum(c, NCH - 1) * CH
        pltpu.async_copy(table_hbm.at[idx_s.at[pl.ds(co, CH)]],
                         sbufs[p], sems_s[p])
        pltpu.async_copy(table_hbm.at[idx_d.at[pl.ds(co, CH)]],
                         dbufs[p], sems_d[p])

    def wait(c, p):
        co = jnp.minimum(c, NCH - 1) * CH
        pltpu.make_async_copy(table_hbm.at[idx_s.at[pl.ds(co, CH)]],
                              sbufs[p], sems_s[p]).wait()
        pltpu.make_async_copy(table_hbm.at[idx_d.at[pl.ds(co, CH)]],
                              dbufs[p], sems_d[p]).wait()

    def compute(p, co):
        buf_s = sbufs[p]
        buf_d = dbufs[p]

        def tup(t, carry):
            for w in range(8):
                lo = w * 16
                acc_e = jnp.zeros((16,), jnp.float32)
                acc_o = jnp.zeros((16,), jnp.float32)
                for b in range(BMAX):
                    sv = plsc.bitcast(buf_s[t, b, pl.ds(lo, 16)],
                                      jnp.bfloat16)
                    dv = plsc.bitcast(buf_d[t, b, pl.ds(lo, 16)],
                                      jnp.bfloat16)
                    pe, po = plsc.unpack(
                        sv * dv, format=plsc.PackFormat.INTERLEAVED)
                    acc_e = acc_e + pe
                    acc_o = acc_o + po
                orows[t, pl.ds(lo, 16)] = acc_e
                orows[t, pl.ds(128 + lo, 16)] = acc_o
            return carry

        lax.fori_loop(0, CH, tup, 0)
        pltpu.sync_copy(orows, out_hbm.at[pl.ds(base + co, CH)])

    fire(0, 0)

    def pair(c2, carry):
        c0 = c2 * 2
        fire(c0 + 1, 1)
        wait(c0, 0)
        compute(0, c0 * CH)
        fire(c0 + 2, 0)
        wait(c0 + 1, 1)
        compute(1, (c0 + 1) * CH)
        return carry

    lax.fori_loop(0, NCH // 2, pair, 0)
    wait(NCH, 0)


@functools.cache
def _sc_gather():
    return pl.kernel(
        _sc_body,
        out_type=jax.ShapeDtypeStruct((T, OUT), jnp.float32),
        mesh=plsc.VectorSubcoreMesh(core_axis_name="c", subcore_axis_name="s"),
        compiler_params=pltpu.CompilerParams(needs_layout_passes=False),
        scratch_types=[
            pltpu.VMEM((TPW,), jnp.int32),
            pltpu.VMEM((TPW,), jnp.int32),
            pltpu.VMEM((CH, BMAX, 128), jnp.int32),
            pltpu.VMEM((CH, BMAX, 128), jnp.int32),
            pltpu.VMEM((CH, BMAX, 128), jnp.int32),
            pltpu.VMEM((CH, BMAX, 128), jnp.int32),
            pltpu.VMEM((CH, OUT), jnp.float32),
            pltpu.SemaphoreType.DMA,
            pltpu.SemaphoreType.DMA,
            pltpu.SemaphoreType.DMA,
            pltpu.SemaphoreType.DMA,
        ],
    )


def kernel(X, adj_t, tuples_coo, W, ln_scale, ln_bias):
    W0 = W[:D]
    wrow = W[D:D + 1]
    deg3, xw, adj_bf = _deg_xw(adj_t, X, W0)
    idx, scales, e = _topk(deg3.reshape(32, 128))
    del idx
    table = _table(scales, adj_bf, xw, e, wrow,
                   ln_scale.reshape(1, OUT), ln_bias.reshape(1, OUT))
    return _sc_gather()(table, tuples_coo.astype(jnp.int32))


# speedup vs baseline: 4.9099x; 1.3571x over previous
import functools

import jax
import jax.numpy as jnp
from jax import lax
from jax.experimental import pallas as pl
from jax.experimental.pallas import tpu as pltpu
from jax.experimental.pallas import tpu_sc as plsc

N = 4096
D = 256
T = 32768
OUT = 256
KSEL = 8
BMAX = 16

BLK = 512
BLKK = 1024
NI = N // BLK
NK = N // BLKK


def _deg_xw_body(adj_ref, x_ref, w0_ref, deg_ref, xw_ref, adjb_ref):
    adj = adj_ref[...]
    deg_ref[0, 0, :] = jnp.sum(adj, axis=1)
    adjb_ref[...] = adj.astype(jnp.bfloat16)
    xw_ref[...] = jnp.dot(x_ref[...], w0_ref[...],
                          preferred_element_type=jnp.float32
                          ).astype(jnp.bfloat16)


def _deg_xw(adj_t, X, W0):
    return pl.pallas_call(
        _deg_xw_body,
        grid=(NI,),
        in_specs=[
            pl.BlockSpec((BLK, N), lambda i: (i, 0)),
            pl.BlockSpec((BLK, D), lambda i: (i, 0)),
            pl.BlockSpec((D, OUT), lambda i: (0, 0)),
        ],
        out_specs=[
            pl.BlockSpec((1, 1, BLK), lambda i: (i, 0, 0)),
            pl.BlockSpec((BLK, OUT), lambda i: (i, 0)),
            pl.BlockSpec((BLK, N), lambda i: (i, 0)),
        ],
        out_shape=[
            jax.ShapeDtypeStruct((NI, 1, BLK), jnp.float32),
            jax.ShapeDtypeStruct((N, OUT), jnp.bfloat16),
            jax.ShapeDtypeStruct((N, N), jnp.bfloat16),
        ],
    )(adj_t, X, W0)


def _topk_body(deg_ref, idx_ref, scale_ref, e_ref):
    d = deg_ref[...]
    gid = (lax.broadcasted_iota(jnp.int32, d.shape, 0) * 128
           + lax.broadcasted_iota(jnp.int32, d.shape, 1))
    cur = d
    vals = []
    for j in range(BMAX):
        m = jnp.max(cur)
        ix = jnp.min(jnp.where(cur == m, gid, jnp.int32(2**30)))
        vals.append(m)
        idx_ref[j] = ix
        cur = jnp.where(gid == ix, -jnp.inf, cur)
    b_count = jnp.int32(KSEL)
    for j in range(KSEL, BMAX):
        b_count = b_count + (vals[j] == vals[KSEL - 1]).astype(jnp.int32)
    inv_sqrt_b = lax.rsqrt(b_count.astype(jnp.float32))
    for b in range(BMAX):
        scale_ref[b] = jnp.where(b < b_count, inv_sqrt_b, 0.0)
    colid = lax.broadcasted_iota(jnp.int32, (1, BMAX), 1)
    idxvec = jnp.zeros((1, BMAX), jnp.int32)
    for b in range(BMAX):
        idxvec = jnp.where(colid == b, idx_ref[b], idxvec)
    rowid = lax.broadcasted_iota(jnp.int32, (N, BMAX), 0)
    e_ref[...] = (rowid == idxvec).astype(jnp.bfloat16)


def _topk(deg):
    return pl.pallas_call(
        _topk_body,
        in_specs=[pl.BlockSpec(memory_space=pltpu.VMEM)],
        out_specs=[
            pl.BlockSpec(memory_space=pltpu.SMEM),
            pl.BlockSpec(memory_space=pltpu.SMEM),
            pl.BlockSpec(memory_space=pltpu.VMEM),
        ],
        out_shape=[
            jax.ShapeDtypeStruct((BMAX,), jnp.int32),
            jax.ShapeDtypeStruct((BMAX,), jnp.float32),
            jax.ShapeDtypeStruct((N, BMAX), jnp.bfloat16),
        ],
    )(deg)


def _pack_words(x):
    lo = lax.bitcast_convert_type(x[:, :128].astype(jnp.bfloat16),
                                  jnp.uint16).astype(jnp.uint32)
    hi = lax.bitcast_convert_type(x[:, 128:].astype(jnp.bfloat16),
                                  jnp.uint16).astype(jnp.uint32)
    return lax.bitcast_convert_type(lo | (hi << 16), jnp.int32)


def _table_body(scale_ref, adj_ref, xw_ref, e_ref, wrow_ref, lns_ref,
                lnb_ref, table_ref, acc_ref, accg_ref):
    k = pl.program_id(1)

    @pl.when(k == 0)
    def _():
        acc_ref[...] = jnp.zeros_like(acc_ref)
        accg_ref[...] = jnp.zeros_like(accg_ref)

    adj = adj_ref[...]
    acc_ref[...] += jnp.dot(adj, xw_ref[...],
                            preferred_element_type=jnp.float32)
    accg_ref[...] += jnp.dot(adj, e_ref[...],
                             preferred_element_type=jnp.float32)

    @pl.when(k == NK - 1)
    def _():
        a = acc_ref[...]
        g = accg_ref[...]
        w = wrow_ref[...]
        lns = lns_ref[...]
        lnb = lnb_ref[...]
        for b in range(BMAX):
            sb = scale_ref[b]
            h = jnp.maximum(a + g[:, b:b + 1] * w, 0.0)
            mu = jnp.mean(h, axis=1, keepdims=True)
            msq = jnp.mean(h * h, axis=1, keepdims=True)
            c1 = lax.rsqrt(msq - mu * mu + 1e-5) * sb
            table_ref[:, b, :] = _pack_words(
                (h - mu) * c1 * lns + lnb * sb)


def _table(scales, adj_bf, xw, e, wrow, lns, lnb):
    return pl.pallas_call(
        _table_body,
        grid=(NI, NK),
        in_specs=[
            pl.BlockSpec(memory_space=pltpu.SMEM),
            pl.BlockSpec((BLK, BLKK), lambda i, k: (i, k)),
            pl.BlockSpec((BLKK, OUT), lambda i, k: (k, 0)),
            pl.BlockSpec((BLKK, BMAX), lambda i, k: (k, 0)),
            pl.BlockSpec((1, OUT), lambda i, k: (0, 0)),
            pl.BlockSpec((1, OUT), lambda i, k: (0, 0)),
            pl.BlockSpec((1, OUT), lambda i, k: (0, 0)),
        ],
        out_specs=pl.BlockSpec((BLK, BMAX, 128), lambda i, k: (i, 0, 0)),
        out_shape=jax.ShapeDtypeStruct((N, BMAX, 128), jnp.int32),
        scratch_shapes=[
            pltpu.VMEM((BLK, OUT), jnp.float32),
            pltpu.VMEM((BLK, BMAX), jnp.float32),
        ],
    )(scales, adj_bf, xw, e, wrow, lns, lnb)


NW = 32
TPW = T // NW
CH = 8
NCH = TPW // CH


def _sc_body(table_hbm, tup_hbm, out_hbm, idx_s, idx_d, sbuf0, sbuf1, dbuf0,
             dbuf1, orows, sem_s0, sem_s1, sem_d0, sem_d1):
    wid = lax.axis_index("s") * 2 + lax.axis_index("c")
    base = wid * TPW
    pltpu.sync_copy(tup_hbm.at[0, pl.ds(base, TPW)], idx_s)
    pltpu.sync_copy(tup_hbm.at[1, pl.ds(base, TPW)], idx_d)

    sbufs = (sbuf0, sbuf1)
    dbufs = (dbuf0, dbuf1)
    sems_s = (sem_s0, sem_s1)
    sems_d = (sem_d0, sem_d1)

    def fire(c, p):
        co = jnp.minimum(c, NCH - 1) * CH
        pltpu.async_copy(table_hbm.at[idx_s.at[pl.ds(co, CH)]],
                         sbufs[p], sems_s[p])
        pltpu.async_copy(table_hbm.at[idx_d.at[pl.ds(co, CH)]],
                         dbufs[p], sems_d[p])

    def wait(c, p):
        co = jnp.minimum(c, NCH - 1) * CH
        pltpu.make_async_copy(table_hbm.at[idx_s.at[pl.ds(co, CH)]],
                              sbufs[p], sems_s[p]).wait()
        pltpu.make_async_copy(table_hbm.at[idx_d.at[pl.ds(co, CH)]],
                              dbufs[p], sems_d[p]).wait()

    def compute(p, co):
        buf_s = sbufs[p]
        buf_d = dbufs[p]

        def tup(t, carry):
            for w in range(8):
                lo = w * 16
                acc_e = jnp.zeros((16,), jnp.float32)
                acc_o = jnp.zeros((16,), jnp.float32)
                for b in range(BMAX):
                    sv = plsc.bitcast(buf_s[t, b, pl.ds(lo, 16)],
                                      jnp.bfloat16)
                    dv = plsc.bitcast(buf_d[t, b, pl.ds(lo, 16)],
                                      jnp.bfloat16)
                    pe, po = plsc.unpack(
                        sv * dv, format=plsc.PackFormat.INTERLEAVED)
                    acc_e = acc_e + pe
                    acc_o = acc_o + po
                orows[t, pl.ds(lo, 16)] = acc_e
                orows[t, pl.ds(128 + lo, 16)] = acc_o
            return carry

        lax.fori_loop(0, CH, tup, 0)
        pltpu.sync_copy(orows, out_hbm.at[pl.ds(base + co, CH)])

    fire(0, 0)

    def pair(c2, carry):
        c0 = c2 * 2
        fire(c0 + 1, 1)
        wait(c0, 0)
        compute(0, c0 * CH)
        fire(c0 + 2, 0)
        wait(c0 + 1, 1)
        compute(1, (c0 + 1) * CH)
        return carry

    lax.fori_loop(0, NCH // 2, pair, 0)
    wait(NCH, 0)


@functools.cache
def _sc_gather():
    return pl.kernel(
        _sc_body,
        out_type=jax.ShapeDtypeStruct((T, OUT), jnp.float32),
        mesh=plsc.VectorSubcoreMesh(core_axis_name="c", subcore_axis_name="s"),
        compiler_params=pltpu.CompilerParams(needs_layout_passes=False),
        scratch_types=[
            pltpu.VMEM((TPW,), jnp.int32),
            pltpu.VMEM((TPW,), jnp.int32),
            pltpu.VMEM((CH, BMAX, 128), jnp.int32),
            pltpu.VMEM((CH, BMAX, 128), jnp.int32),
            pltpu.VMEM((CH, BMAX, 128), jnp.int32),
            pltpu.VMEM((CH, BMAX, 128), jnp.int32),
            pltpu.VMEM((CH, OUT), jnp.float32),
            pltpu.SemaphoreType.DMA,
            pltpu.SemaphoreType.DMA,
            pltpu.SemaphoreType.DMA,
            pltpu.SemaphoreType.DMA,
        ],
    )


def kernel(X, adj_t, tuples_coo, W, ln_scale, ln_bias):
    W0 = W[:D]
    wrow = W[D:D + 1]
    deg3, xw, adj_bf = _deg_xw(adj_t, X, W0)
    idx, scales, e = _topk(deg3.reshape(32, 128))
    del idx
    table = _table(scales, adj_bf, xw, e, wrow,
                   ln_scale.reshape(1, OUT), ln_bias.reshape(1, OUT))
    return _sc_gather()(table, tuples_coo.astype(jnp.int32))
